# Initial kernel scaffold; baseline (speedup 1.0000x reference)
import jax
import jax.numpy as jnp
from jax.experimental import pallas as pl


def kernel(xyz, features, W_d1, b_d1, W_d2, b_d2, W_g1, b_g1, W_g2, b_g2):
    raise NotImplementedError("write your pallas kernel here")


# speedup vs baseline: 8.9606x; 8.9606x over previous
import functools

import jax
import jax.numpy as jnp
from jax import lax
from jax.experimental import pallas as pl
from jax.experimental.pallas import tpu as pltpu
from jax.experimental.pallas import tpu_sc as plsc

K = 16
R = 128
SC_CHUNK = 128
HI = float("inf")


def _knn_body(xyz_ref, xyzT_ref, idx_ref, *, n):
    b = pl.program_id(0)
    xq = xyz_ref[0]
    xT = xyzT_ref[0]
    sq_all = xT[0:1, :] * xT[0:1, :] + xT[1:2, :] * xT[1:2, :] + xT[2:3, :] * xT[2:3, :]
    sq_q = jnp.sum(xq * xq, axis=1, keepdims=True)
    xqb = xq.astype(jnp.bfloat16).astype(jnp.float32)
    xTb = xT.astype(jnp.bfloat16).astype(jnp.float32)
    prod = (xqb[:, 0:1] * xTb[0:1, :] + xqb[:, 1:2] * xTb[1:2, :]) \
        + xqb[:, 2:3] * xTb[2:3, :]
    d = (-2.0 * prod + sq_q) + sq_all

    iota = lax.broadcasted_iota(jnp.int32, (R, n), 1)
    lane_k = lax.broadcasted_iota(jnp.int32, (R, K), 1)
    acc = jnp.zeros((R, K), jnp.int32)
    for k in range(K):
        m = jnp.min(d, axis=1, keepdims=True)
        fi = jnp.min(jnp.where(d == m, iota, n), axis=1)
        acc = jnp.where(lane_k == k, fi[:, None], acc)
        d = jnp.where(iota == fi[:, None], HI, d)
    idx_ref[0] = acc + b * n


def _knn_indices(xyz, xyzT):
    bsz, n, _ = xyz.shape
    grid = (bsz, n // R)
    return pl.pallas_call(
        functools.partial(_knn_body, n=n),
        grid=grid,
        in_specs=[
            pl.BlockSpec((1, R, 3), lambda b, r: (b, r, 0)),
            pl.BlockSpec((1, 3, n), lambda b, r: (b, 0, 0)),
        ],
        out_specs=pl.BlockSpec((1, R, K), lambda b, r: (b, r, 0)),
        out_shape=jax.ShapeDtypeStruct((bsz, n, K), jnp.int32),
    )(xyz, xyzT)


def _sc_gather(feat_flat, xyzp_1d, idx_flat, n):
    m, d = idx_flat.shape[0], feat_flat.shape[1]
    info = plsc.get_sparse_core_info()
    nw = info.num_cores * info.num_subcores
    per_w = m // nw
    nchunk = per_w // SC_CHUNK
    mesh = plsc.VectorSubcoreMesh(core_axis_name="c", subcore_axis_name="s")

    @functools.partial(
        pl.kernel,
        mesh=mesh,
        compiler_params=pltpu.CompilerParams(needs_layout_passes=False),
        out_type=(
            jax.ShapeDtypeStruct((m, d), jnp.float32),
            jax.ShapeDtypeStruct((m * 16,), jnp.float32),
        ),
        scratch_types=[
            pltpu.VMEM((SC_CHUNK,), jnp.int32),
            pltpu.VMEM((SC_CHUNK, d), jnp.float32),
            pltpu.VMEM((SC_CHUNK * 16,), jnp.float32),
            pltpu.VMEM((n * 16,), jnp.float32),
            pltpu.SemaphoreType.DMA,
        ],
    )
    def gather_k(feat_hbm, xyzp_hbm, idx_hbm, fout, xout,
                 idx_v, frows, xrows, xtab, sf):
        wid = lax.axis_index("s") * info.num_cores + lax.axis_index("c")
        base0 = wid * per_w
        b = base0 // (n * K)
        pltpu.sync_copy(xyzp_hbm.at[pl.ds(b * n * 16, n * 16)], xtab)
        iota16 = lax.broadcasted_iota(jnp.int32, (16,), 0)

        def body(i, carry):
            base = base0 + i * SC_CHUNK
            pltpu.sync_copy(idx_hbm.at[pl.ds(base, SC_CHUNK)], idx_v)
            cf = pltpu.async_copy(feat_hbm.at[idx_v], frows, sf)
            for g in range(SC_CHUNK // 16):
                jloc = idx_v[pl.ds(g * 16, 16)] - b * n
                row_base = jloc * 16
                for c in range(16):
                    vals = plsc.load_gather(xtab, [row_base + c])
                    plsc.store_scatter(
                        xrows, [iota16 * 16 + (g * 256 + c)], vals)
            cf.wait()
            pltpu.sync_copy(frows, fout.at[pl.ds(base, SC_CHUNK)])
            pltpu.sync_copy(xrows, xout.at[pl.ds(base * 16, SC_CHUNK * 16)])
            return carry

        lax.fori_loop(0, nchunk, body, 0)

    return gather_k(feat_flat, xyzp_1d, idx_flat)


def _mlp_body(feat_ref, xyz_ref, kf_ref, kx_ref,
              wd1_ref, bd1_ref, wd2_ref, bd2_ref,
              wg1_ref, bg1_ref, wg2_ref, bg2_ref, out_ref):
    d = feat_ref.shape[-1]
    kf = kf_ref[0].reshape(R * K, d)
    kx = kx_ref[0][:, :, 0:3]
    xq = xyz_ref[0]
    delta = (xq[:, None, :] - kx).reshape(R * K, 3)

    hp = jnp.dot(delta, wd1_ref[...], preferred_element_type=jnp.float32,
                 precision=lax.Precision.HIGHEST) + bd1_ref[...]
    h = jnp.maximum(hp, 0.0)
    de = jnp.dot(h, wd2_ref[...], preferred_element_type=jnp.float32,
                 precision=lax.Precision.HIGHEST) + bd2_ref[...]
    xk = kf + de

    gp = jnp.dot(xk, wg1_ref[...], preferred_element_type=jnp.float32,
                 precision=lax.Precision.HIGHEST) + bg1_ref[...]
    g = jnp.maximum(gp, 0.0)
    w = jnp.dot(g, wg2_ref[...], preferred_element_type=jnp.float32,
                precision=lax.Precision.HIGHEST) + bg2_ref[...]

    w3 = w.reshape(R, K, d)
    mx = jnp.max(w3, axis=1, keepdims=True)
    e = jnp.exp(w3 - mx)
    sm = e / jnp.sum(e, axis=1, keepdims=True)
    res = jnp.sum(sm * xk.reshape(R, K, d), axis=1)
    out_ref[0] = feat_ref[0] + res


def _mlp(features, xyz, kfeat, kxyz, wd1t, bd1, wd2t, bd2, wg1t, bg1, wg2t, bg2):
    bsz, n, d = features.shape
    grid = (bsz, n // R)
    return pl.pallas_call(
        _mlp_body,
        grid=grid,
        in_specs=[
            pl.BlockSpec((1, R, d), lambda b, r: (b, r, 0)),
            pl.BlockSpec((1, R, 3), lambda b, r: (b, r, 0)),
            pl.BlockSpec((1, R, K, d), lambda b, r: (b, r, 0, 0)),
            pl.BlockSpec((1, R, K, 16), lambda b, r: (b, r, 0, 0)),
            pl.BlockSpec(wd1t.shape, lambda b, r: (0, 0)),
            pl.BlockSpec(bd1.shape, lambda b, r: (0, 0)),
            pl.BlockSpec(wd2t.shape, lambda b, r: (0, 0)),
            pl.BlockSpec(bd2.shape, lambda b, r: (0, 0)),
            pl.BlockSpec(wg1t.shape, lambda b, r: (0, 0)),
            pl.BlockSpec(bg1.shape, lambda b, r: (0, 0)),
            pl.BlockSpec(wg2t.shape, lambda b, r: (0, 0)),
            pl.BlockSpec(bg2.shape, lambda b, r: (0, 0)),
        ],
        out_specs=pl.BlockSpec((1, R, d), lambda b, r: (b, r, 0)),
        out_shape=jax.ShapeDtypeStruct((bsz, n, d), jnp.float32),
    )(features, xyz, kfeat, kxyz, wd1t, bd1, wd2t, bd2, wg1t, bg1, wg2t, bg2)


def kernel(xyz, features, W_d1, b_d1, W_d2, b_d2, W_g1, b_g1, W_g2, b_g2):
    bsz, n, _ = xyz.shape
    d = features.shape[-1]

    xyzT = jnp.transpose(xyz, (0, 2, 1))
    idx = _knn_indices(xyz, xyzT)
    idx_flat = idx.reshape(bsz * n * K)

    feat_flat = features.reshape(bsz * n, d)
    xyzp_1d = jnp.pad(xyz, ((0, 0), (0, 0), (0, 13))).reshape(bsz * n * 16)
    kfeat, kxyz = _sc_gather(feat_flat, xyzp_1d, idx_flat, n)

    kfeat = kfeat.reshape(bsz, n, K, d)
    kxyz = kxyz.reshape(bsz, n, K, 16)
    return _mlp(
        features, xyz, kfeat, kxyz,
        W_d1.T, b_d1.reshape(1, d), W_d2.T, b_d2.reshape(1, d),
        W_g1.T, b_g1.reshape(1, d), W_g2.T, b_g2.reshape(1, d),
    )


# speedup vs baseline: 16.7261x; 1.8666x over previous
import functools

import jax
import jax.numpy as jnp
from jax import lax
from jax.experimental import pallas as pl
from jax.experimental.pallas import tpu as pltpu
from jax.experimental.pallas import tpu_sc as plsc

K = 16
R = 128
SC_CHUNK = 128
HI = float("inf")


def _knn_body(xyz_ref, xyzT_ref, w1t_ref, idx_ref, a_ref, *, n):
    b = pl.program_id(0)
    xq = xyz_ref[0]
    xT = xyzT_ref[0]
    sq_all = xT[0:1, :] * xT[0:1, :] + xT[1:2, :] * xT[1:2, :] + xT[2:3, :] * xT[2:3, :]
    sq_q = jnp.sum(xq * xq, axis=1, keepdims=True)
    xqb = xq.astype(jnp.bfloat16).astype(jnp.float32)
    xTb = xT.astype(jnp.bfloat16).astype(jnp.float32)
    prod = (xqb[:, 0:1] * xTb[0:1, :] + xqb[:, 1:2] * xTb[1:2, :]) \
        + xqb[:, 2:3] * xTb[2:3, :]
    d = (-2.0 * prod + sq_q) + sq_all

    iota = lax.broadcasted_iota(jnp.int32, (R, n), 1)
    lane_k = lax.broadcasted_iota(jnp.int32, (R, K), 1)
    acc = jnp.zeros((R, K), jnp.int32)
    for k in range(K):
        m = jnp.min(d, axis=1, keepdims=True)
        eq = d == m
        fi = jnp.min(jnp.where(eq, iota, n), axis=1)
        acc = jnp.where(lane_k == k, fi[:, None], acc)
        d = jnp.where(eq, HI, d)
    idx_ref[0] = acc + b * n

    a_ref[0] = jnp.dot(xq.astype(jnp.bfloat16), w1t_ref[...].astype(jnp.bfloat16),
                       preferred_element_type=jnp.float32)


def _knn_indices(xyz, xyzT, w1t):
    bsz, n, _ = xyz.shape
    d = w1t.shape[1]
    grid = (bsz, n // R)
    return pl.pallas_call(
        functools.partial(_knn_body, n=n),
        grid=grid,
        in_specs=[
            pl.BlockSpec((1, R, 3), lambda b, r: (b, r, 0)),
            pl.BlockSpec((1, 3, n), lambda b, r: (b, 0, 0)),
            pl.BlockSpec(w1t.shape, lambda b, r: (0, 0)),
        ],
        out_specs=[
            pl.BlockSpec((1, R, K), lambda b, r: (b, r, 0)),
            pl.BlockSpec((1, R, d), lambda b, r: (b, r, 0)),
        ],
        out_shape=[
            jax.ShapeDtypeStruct((bsz, n, K), jnp.int32),
            jax.ShapeDtypeStruct((bsz, n, d), jnp.float32),
        ],
    )(xyz, xyzT, w1t)


def _sc_gather(feat_flat, a_flat, idx_flat):
    m, d = idx_flat.shape[0], feat_flat.shape[1]
    info = plsc.get_sparse_core_info()
    nw = info.num_cores * info.num_subcores
    per_w = m // nw
    nchunk = per_w // SC_CHUNK
    mesh = plsc.VectorSubcoreMesh(core_axis_name="c", subcore_axis_name="s")

    @functools.partial(
        pl.kernel,
        mesh=mesh,
        compiler_params=pltpu.CompilerParams(needs_layout_passes=False),
        out_type=(
            jax.ShapeDtypeStruct((m, d), jnp.float32),
            jax.ShapeDtypeStruct((m, d), jnp.float32),
        ),
        scratch_types=[
            pltpu.VMEM((SC_CHUNK,), jnp.int32),
            pltpu.VMEM((SC_CHUNK, d), jnp.float32),
            pltpu.VMEM((SC_CHUNK, d), jnp.float32),
            pltpu.SemaphoreType.DMA,
            pltpu.SemaphoreType.DMA,
        ],
    )
    def gather_k(feat_hbm, a_hbm, idx_hbm, fout, aout,
                 idx_v, frows, arows, sf, sa):
        wid = lax.axis_index("s") * info.num_cores + lax.axis_index("c")
        base0 = wid * per_w

        def body(i, carry):
            base = base0 + i * SC_CHUNK
            pltpu.sync_copy(idx_hbm.at[pl.ds(base, SC_CHUNK)], idx_v)
            cf = pltpu.async_copy(feat_hbm.at[idx_v], frows, sf)
            ca = pltpu.async_copy(a_hbm.at[idx_v], arows, sa)
            cf.wait()
            ca.wait()
            pltpu.sync_copy(frows, fout.at[pl.ds(base, SC_CHUNK)])
            pltpu.sync_copy(arows, aout.at[pl.ds(base, SC_CHUNK)])
            return carry

        lax.fori_loop(0, nchunk, body, 0)

    return gather_k(feat_flat, a_flat, idx_flat)


def _mlp_body(feat_ref, aq_ref, kf_ref, ka_ref,
              bd1_ref, wd2_ref, bd2_ref,
              wg1_ref, bg1_ref, wg2_ref, bg2_ref, out_ref):
    d = feat_ref.shape[-1]
    kf = kf_ref[0].reshape(R * K, d)
    ka = ka_ref[0].reshape(R * K, d)
    aq = aq_ref[0][:, None, :]
    aq = jnp.broadcast_to(aq, (R, K, d)).reshape(R * K, d)

    h = jnp.maximum((aq - ka) + bd1_ref[...], 0.0)
    de = jnp.dot(h.astype(jnp.bfloat16), wd2_ref[...],
                 preferred_element_type=jnp.float32) + bd2_ref[...]
    xk = kf + de

    gp = jnp.dot(xk.astype(jnp.bfloat16), wg1_ref[...],
                 preferred_element_type=jnp.float32) + bg1_ref[...]
    g = jnp.maximum(gp, 0.0)
    w = jnp.dot(g.astype(jnp.bfloat16), wg2_ref[...],
                preferred_element_type=jnp.float32) + bg2_ref[...]

    w3 = w.reshape(R, K, d)
    mx = jnp.max(w3, axis=1, keepdims=True)
    e = jnp.exp(w3 - mx)
    sm = e / jnp.sum(e, axis=1, keepdims=True)
    res = jnp.sum(sm * xk.reshape(R, K, d), axis=1)
    out_ref[0] = feat_ref[0] + res


def _mlp(features, aq, kfeat, ka, bd1, wd2t, bd2, wg1t, bg1, wg2t, bg2):
    bsz, n, d = features.shape
    grid = (bsz, n // R)
    return pl.pallas_call(
        _mlp_body,
        grid=grid,
        in_specs=[
            pl.BlockSpec((1, R, d), lambda b, r: (b, r, 0)),
            pl.BlockSpec((1, R, d), lambda b, r: (b, r, 0)),
            pl.BlockSpec((1, R, K, d), lambda b, r: (b, r, 0, 0)),
            pl.BlockSpec((1, R, K, d), lambda b, r: (b, r, 0, 0)),
            pl.BlockSpec(bd1.shape, lambda b, r: (0, 0)),
            pl.BlockSpec(wd2t.shape, lambda b, r: (0, 0)),
            pl.BlockSpec(bd2.shape, lambda b, r: (0, 0)),
            pl.BlockSpec(wg1t.shape, lambda b, r: (0, 0)),
            pl.BlockSpec(bg1.shape, lambda b, r: (0, 0)),
            pl.BlockSpec(wg2t.shape, lambda b, r: (0, 0)),
            pl.BlockSpec(bg2.shape, lambda b, r: (0, 0)),
        ],
        out_specs=pl.BlockSpec((1, R, d), lambda b, r: (b, r, 0)),
        out_shape=jax.ShapeDtypeStruct((bsz, n, d), jnp.float32),
    )(features, aq, kfeat, ka, bd1, wd2t, bd2, wg1t, bg1, wg2t, bg2)


def kernel(xyz, features, W_d1, b_d1, W_d2, b_d2, W_g1, b_g1, W_g2, b_g2):
    bsz, n, _ = xyz.shape
    d = features.shape[-1]

    xyzT = jnp.transpose(xyz, (0, 2, 1))
    idx, aq = _knn_indices(xyz, xyzT, W_d1.T)
    idx_flat = idx.reshape(bsz * n * K)

    feat_flat = features.reshape(bsz * n, d)
    a_flat = aq.reshape(bsz * n, d)
    kfeat, ka = _sc_gather(feat_flat, a_flat, idx_flat)

    kfeat = kfeat.reshape(bsz, n, K, d)
    ka = ka.reshape(bsz, n, K, d)
    bf = jnp.bfloat16
    return _mlp(
        features, aq, kfeat, ka,
        b_d1.reshape(1, d), W_d2.T.astype(bf), b_d2.reshape(1, d),
        W_g1.T.astype(bf), b_g1.reshape(1, d), W_g2.T.astype(bf), b_g2.reshape(1, d),
    )


# speedup vs baseline: 22.1596x; 1.3249x over previous
import functools

import jax
import jax.numpy as jnp
from jax import lax
from jax.experimental import pallas as pl
from jax.experimental.pallas import tpu as pltpu
from jax.experimental.pallas import tpu_sc as plsc

K = 16
R = 128
SC_CHUNK = 128
HI = float("inf")


def _knn_body(xyz_ref, xyzT_ref, w1t_ref, idx_ref, a_ref, *, n):
    b = pl.program_id(0)
    xq = xyz_ref[0]
    xT = xyzT_ref[0]
    sq_all = xT[0:1, :] * xT[0:1, :] + xT[1:2, :] * xT[1:2, :] + xT[2:3, :] * xT[2:3, :]
    sq_q = jnp.sum(xq * xq, axis=1, keepdims=True)
    xqb = xq.astype(jnp.bfloat16).astype(jnp.float32)
    xTb = xT.astype(jnp.bfloat16).astype(jnp.float32)
    prod = (xqb[:, 0:1] * xTb[0:1, :] + xqb[:, 1:2] * xTb[1:2, :]) \
        + xqb[:, 2:3] * xTb[2:3, :]
    d = (-2.0 * prod + sq_q) + sq_all

    lane_k = lax.broadcasted_iota(jnp.int32, (R, K), 1)
    iota_f = lax.broadcasted_iota(jnp.int32, (R, n), 1).astype(jnp.float32)
    acc = jnp.zeros((R, K), jnp.int32)
    for k in range(K):
        m = jnp.min(d, axis=1, keepdims=True)
        eq = d == m
        fi_f = jnp.min(jnp.where(eq, iota_f, HI), axis=1, keepdims=True)
        acc = jnp.where(lane_k == k, fi_f.astype(jnp.int32), acc)
        d = jnp.where(eq, HI, d)
    idx_ref[0] = acc + b * n

    a_ref[0] = jnp.dot(xq.astype(jnp.bfloat16), w1t_ref[...].astype(jnp.bfloat16),
                       preferred_element_type=jnp.float32)


def _knn_indices(xyz, xyzT, w1t):
    bsz, n, _ = xyz.shape
    d = w1t.shape[1]
    grid = (bsz, n // R)
    return pl.pallas_call(
        functools.partial(_knn_body, n=n),
        grid=grid,
        in_specs=[
            pl.BlockSpec((1, R, 3), lambda b, r: (b, r, 0)),
            pl.BlockSpec((1, 3, n), lambda b, r: (b, 0, 0)),
            pl.BlockSpec(w1t.shape, lambda b, r: (0, 0)),
        ],
        out_specs=[
            pl.BlockSpec((1, R, K), lambda b, r: (b, r, 0)),
            pl.BlockSpec((1, R, d), lambda b, r: (b, r, 0)),
        ],
        out_shape=[
            jax.ShapeDtypeStruct((bsz, n, K), jnp.int32),
            jax.ShapeDtypeStruct((bsz, n, d), jnp.float32),
        ],
    )(xyz, xyzT, w1t)


def _sc_gather(feat_flat, a_flat, idx_flat):
    m, d = idx_flat.shape[0], feat_flat.shape[1]
    info = plsc.get_sparse_core_info()
    nw = info.num_cores * info.num_subcores
    per_w = m // nw
    nchunk = per_w // SC_CHUNK
    mesh = plsc.VectorSubcoreMesh(core_axis_name="c", subcore_axis_name="s")

    @functools.partial(
        pl.kernel,
        mesh=mesh,
        compiler_params=pltpu.CompilerParams(needs_layout_passes=False),
        out_type=(
            jax.ShapeDtypeStruct((m, d), jnp.float32),
            jax.ShapeDtypeStruct((m, d), jnp.float32),
        ),
        scratch_types=[
            pltpu.VMEM((SC_CHUNK,), jnp.int32),
            pltpu.VMEM((SC_CHUNK, d), jnp.float32),
            pltpu.VMEM((SC_CHUNK, d), jnp.float32),
            pltpu.SemaphoreType.DMA,
            pltpu.SemaphoreType.DMA,
        ],
    )
    def gather_k(feat_hbm, a_hbm, idx_hbm, fout, aout,
                 idx_v, frows, arows, sf, sa):
        wid = lax.axis_index("s") * info.num_cores + lax.axis_index("c")
        base0 = wid * per_w

        def body(i, carry):
            base = base0 + i * SC_CHUNK
            pltpu.sync_copy(idx_hbm.at[pl.ds(base, SC_CHUNK)], idx_v)
            cf = pltpu.async_copy(feat_hbm.at[idx_v], frows, sf)
            ca = pltpu.async_copy(a_hbm.at[idx_v], arows, sa)
            cf.wait()
            ca.wait()
            pltpu.sync_copy(frows, fout.at[pl.ds(base, SC_CHUNK)])
            pltpu.sync_copy(arows, aout.at[pl.ds(base, SC_CHUNK)])
            return carry

        lax.fori_loop(0, nchunk, body, 0)

    return gather_k(feat_flat, a_flat, idx_flat)


def _mlp_body(feat_ref, aq_ref, kf_ref, ka_ref,
              bd1_ref, wd2_ref, bd2_ref,
              wg1_ref, bg1_ref, wg2_ref, bg2_ref, out_ref):
    d = feat_ref.shape[-1]
    kf = kf_ref[0].reshape(R * K, d)
    ka = ka_ref[0].reshape(R * K, d)
    aq = aq_ref[0][:, None, :]
    aq = jnp.broadcast_to(aq, (R, K, d)).reshape(R * K, d)

    h = jnp.maximum((aq - ka) + bd1_ref[...], 0.0)
    de = jnp.dot(h.astype(jnp.bfloat16), wd2_ref[...],
                 preferred_element_type=jnp.float32) + bd2_ref[...]
    xk = kf + de

    gp = jnp.dot(xk.astype(jnp.bfloat16), wg1_ref[...],
                 preferred_element_type=jnp.float32) + bg1_ref[...]
    g = jnp.maximum(gp, 0.0)
    w = jnp.dot(g.astype(jnp.bfloat16), wg2_ref[...],
                preferred_element_type=jnp.float32) + bg2_ref[...]

    w3 = w.reshape(R, K, d)
    mx = jnp.max(w3, axis=1, keepdims=True)
    e = jnp.exp(w3 - mx)
    sm = e / jnp.sum(e, axis=1, keepdims=True)
    res = jnp.sum(sm * xk.reshape(R, K, d), axis=1)
    out_ref[0] = feat_ref[0] + res


def _mlp(features, aq, kfeat, ka, bd1, wd2t, bd2, wg1t, bg1, wg2t, bg2):
    bsz, n, d = features.shape
    grid = (bsz, n // R)
    return pl.pallas_call(
        _mlp_body,
        grid=grid,
        in_specs=[
            pl.BlockSpec((1, R, d), lambda b, r: (b, r, 0)),
            pl.BlockSpec((1, R, d), lambda b, r: (b, r, 0)),
            pl.BlockSpec((1, R, K, d), lambda b, r: (b, r, 0, 0)),
            pl.BlockSpec((1, R, K, d), lambda b, r: (b, r, 0, 0)),
            pl.BlockSpec(bd1.shape, lambda b, r: (0, 0)),
            pl.BlockSpec(wd2t.shape, lambda b, r: (0, 0)),
            pl.BlockSpec(bd2.shape, lambda b, r: (0, 0)),
            pl.BlockSpec(wg1t.shape, lambda b, r: (0, 0)),
            pl.BlockSpec(bg1.shape, lambda b, r: (0, 0)),
            pl.BlockSpec(wg2t.shape, lambda b, r: (0, 0)),
            pl.BlockSpec(bg2.shape, lambda b, r: (0, 0)),
        ],
        out_specs=pl.BlockSpec((1, R, d), lambda b, r: (b, r, 0)),
        out_shape=jax.ShapeDtypeStruct((bsz, n, d), jnp.float32),
    )(features, aq, kfeat, ka, bd1, wd2t, bd2, wg1t, bg1, wg2t, bg2)


NSPLIT = 2


def kernel(xyz, features, W_d1, b_d1, W_d2, b_d2, W_g1, b_g1, W_g2, b_g2):
    bsz, n, _ = xyz.shape
    d = features.shape[-1]

    w1t = W_d1.T
    bf = jnp.bfloat16
    mlp_consts = (
        b_d1.reshape(1, d), W_d2.T.astype(bf), b_d2.reshape(1, d),
        W_g1.T.astype(bf), b_g1.reshape(1, d), W_g2.T.astype(bf),
        b_g2.reshape(1, d),
    )

    outs = []
    bh = bsz // NSPLIT
    for h in range(NSPLIT):
        xyz_h = xyz[h * bh:(h + 1) * bh]
        feat_h = features[h * bh:(h + 1) * bh]
        xyzT_h = jnp.transpose(xyz_h, (0, 2, 1))
        idx, aq = _knn_indices(xyz_h, xyzT_h, w1t)
        idx_flat = idx.reshape(bh * n * K)

        feat_flat = feat_h.reshape(bh * n, d)
        a_flat = aq.reshape(bh * n, d)
        kfeat, ka = _sc_gather(feat_flat, a_flat, idx_flat)

        kfeat = kfeat.reshape(bh, n, K, d)
        ka = ka.reshape(bh, n, K, d)
        outs.append(_mlp(feat_h, aq, kfeat, ka, *mlp_consts))
    return jnp.concatenate(outs, axis=0)


# speedup vs baseline: 22.6186x; 1.0207x over previous
import functools

import jax
import jax.numpy as jnp
from jax import lax
from jax.experimental import pallas as pl
from jax.experimental.pallas import tpu as pltpu
from jax.experimental.pallas import tpu_sc as plsc

K = 16
R = 128
SC_CHUNK = 128
HI = float("inf")


def _knn_body(xyz_ref, xyzT_ref, w1t_ref, idx_ref, a_ref, *, n):
    b = pl.program_id(0)
    xq = xyz_ref[0]
    xT = xyzT_ref[0]
    sq_all = xT[0:1, :] * xT[0:1, :] + xT[1:2, :] * xT[1:2, :] + xT[2:3, :] * xT[2:3, :]
    sq_q = jnp.sum(xq * xq, axis=1, keepdims=True)
    xqb = xq.astype(jnp.bfloat16).astype(jnp.float32)
    xTb = xT.astype(jnp.bfloat16).astype(jnp.float32)
    prod = (xqb[:, 0:1] * xTb[0:1, :] + xqb[:, 1:2] * xTb[1:2, :]) \
        + xqb[:, 2:3] * xTb[2:3, :]
    d = (-2.0 * prod + sq_q) + sq_all

    lane_k = lax.broadcasted_iota(jnp.int32, (R, K), 1)
    iota_f = lax.broadcasted_iota(jnp.int32, (R, n), 1).astype(jnp.float32)
    acc = jnp.zeros((R, K), jnp.int32)
    for k in range(K):
        m = jnp.min(d, axis=1, keepdims=True)
        eq = d == m
        fi_f = jnp.min(jnp.where(eq, iota_f, HI), axis=1, keepdims=True)
        acc = jnp.where(lane_k == k, fi_f.astype(jnp.int32), acc)
        d = jnp.where(eq, HI, d)
    idx_ref[0] = acc + b * n

    a_ref[0] = jnp.dot(xq.astype(jnp.bfloat16), w1t_ref[...].astype(jnp.bfloat16),
                       preferred_element_type=jnp.float32)


def _knn_indices(xyz, xyzT, w1t):
    bsz, n, _ = xyz.shape
    d = w1t.shape[1]
    grid = (bsz, n // R)
    return pl.pallas_call(
        functools.partial(_knn_body, n=n),
        grid=grid,
        in_specs=[
            pl.BlockSpec((1, R, 3), lambda b, r: (b, r, 0)),
            pl.BlockSpec((1, 3, n), lambda b, r: (b, 0, 0)),
            pl.BlockSpec(w1t.shape, lambda b, r: (0, 0)),
        ],
        out_specs=[
            pl.BlockSpec((1, R, K), lambda b, r: (b, r, 0)),
            pl.BlockSpec((1, R, d), lambda b, r: (b, r, 0)),
        ],
        out_shape=[
            jax.ShapeDtypeStruct((bsz, n, K), jnp.int32),
            jax.ShapeDtypeStruct((bsz, n, d), jnp.float32),
        ],
    )(xyz, xyzT, w1t)


def _sc_gather(feat_flat, a_flat, idx_flat):
    m, d = idx_flat.shape[0], feat_flat.shape[1]
    info = plsc.get_sparse_core_info()
    nw = info.num_cores * info.num_subcores
    per_w = m // nw
    nchunk = per_w // SC_CHUNK
    mesh = plsc.VectorSubcoreMesh(core_axis_name="c", subcore_axis_name="s")

    @functools.partial(
        pl.kernel,
        mesh=mesh,
        compiler_params=pltpu.CompilerParams(needs_layout_passes=False),
        out_type=(
            jax.ShapeDtypeStruct((m, d), jnp.float32),
            jax.ShapeDtypeStruct((m, d), jnp.float32),
        ),
        scratch_types=[
            pltpu.VMEM((SC_CHUNK,), jnp.int32),
            pltpu.VMEM((SC_CHUNK, d), jnp.float32),
            pltpu.VMEM((SC_CHUNK, d), jnp.float32),
            pltpu.SemaphoreType.DMA,
            pltpu.SemaphoreType.DMA,
        ],
    )
    def gather_k(feat_hbm, a_hbm, idx_hbm, fout, aout,
                 idx_v, frows, arows, sf, sa):
        wid = lax.axis_index("s") * info.num_cores + lax.axis_index("c")
        base0 = wid * per_w

        def body(i, carry):
            base = base0 + i * SC_CHUNK
            pltpu.sync_copy(idx_hbm.at[pl.ds(base, SC_CHUNK)], idx_v)
            cf = pltpu.async_copy(feat_hbm.at[idx_v], frows, sf)
            ca = pltpu.async_copy(a_hbm.at[idx_v], arows, sa)
            cf.wait()
            ca.wait()
            pltpu.sync_copy(frows, fout.at[pl.ds(base, SC_CHUNK)])
            pltpu.sync_copy(arows, aout.at[pl.ds(base, SC_CHUNK)])
            return carry

        lax.fori_loop(0, nchunk, body, 0)

    return gather_k(feat_flat, a_flat, idx_flat)


def _mlp_body(feat_ref, aq_ref, kf_ref, ka_ref,
              bd1_ref, wd2_ref, bd2_ref,
              wg1_ref, bg1_ref, wg2_ref, bg2_ref, out_ref):
    d = feat_ref.shape[-1]
    kf = kf_ref[0].reshape(R * K, d)
    ka = ka_ref[0].reshape(R * K, d)
    aq = aq_ref[0][:, None, :]
    aq = jnp.broadcast_to(aq, (R, K, d)).reshape(R * K, d)

    h = jnp.maximum((aq - ka) + bd1_ref[...], 0.0)
    de = jnp.dot(h.astype(jnp.bfloat16), wd2_ref[...],
                 preferred_element_type=jnp.float32) + bd2_ref[...]
    xk = kf + de

    gp = jnp.dot(xk.astype(jnp.bfloat16), wg1_ref[...],
                 preferred_element_type=jnp.float32) + bg1_ref[...]
    g = jnp.maximum(gp, 0.0)
    w = jnp.dot(g.astype(jnp.bfloat16), wg2_ref[...],
                preferred_element_type=jnp.float32) + bg2_ref[...]

    w3 = w.reshape(R, K, d)
    mx = jnp.max(w3, axis=1, keepdims=True)
    e = jnp.exp(w3 - mx)
    sm = e / jnp.sum(e, axis=1, keepdims=True)
    res = jnp.sum(sm * xk.reshape(R, K, d), axis=1)
    out_ref[0] = feat_ref[0] + res


def _mlp(features, aq, kfeat, ka, bd1, wd2t, bd2, wg1t, bg1, wg2t, bg2):
    bsz, n, d = features.shape
    grid = (bsz, n // R)
    return pl.pallas_call(
        _mlp_body,
        grid=grid,
        in_specs=[
            pl.BlockSpec((1, R, d), lambda b, r: (b, r, 0)),
            pl.BlockSpec((1, R, d), lambda b, r: (b, r, 0)),
            pl.BlockSpec((1, R, K, d), lambda b, r: (b, r, 0, 0)),
            pl.BlockSpec((1, R, K, d), lambda b, r: (b, r, 0, 0)),
            pl.BlockSpec(bd1.shape, lambda b, r: (0, 0)),
            pl.BlockSpec(wd2t.shape, lambda b, r: (0, 0)),
            pl.BlockSpec(bd2.shape, lambda b, r: (0, 0)),
            pl.BlockSpec(wg1t.shape, lambda b, r: (0, 0)),
            pl.BlockSpec(bg1.shape, lambda b, r: (0, 0)),
            pl.BlockSpec(wg2t.shape, lambda b, r: (0, 0)),
            pl.BlockSpec(bg2.shape, lambda b, r: (0, 0)),
        ],
        out_specs=pl.BlockSpec((1, R, d), lambda b, r: (b, r, 0)),
        out_shape=jax.ShapeDtypeStruct((bsz, n, d), jnp.float32),
    )(features, aq, kfeat, ka, bd1, wd2t, bd2, wg1t, bg1, wg2t, bg2)


NSPLIT = 4


def kernel(xyz, features, W_d1, b_d1, W_d2, b_d2, W_g1, b_g1, W_g2, b_g2):
    bsz, n, _ = xyz.shape
    d = features.shape[-1]

    w1t = W_d1.T
    bf = jnp.bfloat16
    mlp_consts = (
        b_d1.reshape(1, d), W_d2.T.astype(bf), b_d2.reshape(1, d),
        W_g1.T.astype(bf), b_g1.reshape(1, d), W_g2.T.astype(bf),
        b_g2.reshape(1, d),
    )

    outs = []
    bh = bsz // NSPLIT
    for h in range(NSPLIT):
        xyz_h = xyz[h * bh:(h + 1) * bh]
        feat_h = features[h * bh:(h + 1) * bh]
        xyzT_h = jnp.transpose(xyz_h, (0, 2, 1))
        idx, aq = _knn_indices(xyz_h, xyzT_h, w1t)
        idx_flat = idx.reshape(bh * n * K)

        feat_flat = feat_h.reshape(bh * n, d)
        a_flat = aq.reshape(bh * n, d)
        kfeat, ka = _sc_gather(feat_flat, a_flat, idx_flat)

        kfeat = kfeat.reshape(bh, n, K, d)
        ka = ka.reshape(bh, n, K, d)
        outs.append(_mlp(feat_h, aq, kfeat, ka, *mlp_consts))
    return jnp.concatenate(outs, axis=0)


# speedup vs baseline: 22.8531x; 1.0104x over previous
import functools

import jax
import jax.numpy as jnp
from jax import lax
from jax.experimental import pallas as pl
from jax.experimental.pallas import tpu as pltpu
from jax.experimental.pallas import tpu_sc as plsc

K = 16
R = 128
SC_CHUNK = 128
HI = float("inf")


def _knn_body(xyz_ref, xyzT_ref, w1t_ref, idx_ref, a_ref, *, n):
    b = pl.program_id(0)
    xq = xyz_ref[0]
    xT = xyzT_ref[0]
    sq_all = xT[0:1, :] * xT[0:1, :] + xT[1:2, :] * xT[1:2, :] + xT[2:3, :] * xT[2:3, :]
    sq_q = jnp.sum(xq * xq, axis=1, keepdims=True)
    prod = jnp.dot(xq.astype(jnp.bfloat16), xT.astype(jnp.bfloat16),
                   preferred_element_type=jnp.float32)
    d = (-2.0 * prod + sq_q) + sq_all

    lane_k = lax.broadcasted_iota(jnp.int32, (R, K), 1)
    iota_f = lax.broadcasted_iota(jnp.int32, (R, n), 1).astype(jnp.float32)
    acc = jnp.zeros((R, K), jnp.int32)
    for k in range(K):
        m = jnp.min(d, axis=1, keepdims=True)
        eq = d == m
        fi_f = jnp.min(jnp.where(eq, iota_f, HI), axis=1, keepdims=True)
        acc = jnp.where(lane_k == k, fi_f.astype(jnp.int32), acc)
        d = jnp.where(eq, HI, d)
    idx_ref[0] = acc + b * n

    a_ref[0] = jnp.dot(xq.astype(jnp.bfloat16), w1t_ref[...].astype(jnp.bfloat16),
                       preferred_element_type=jnp.float32)


def _knn_indices(xyz, xyzT, w1t):
    bsz, n, _ = xyz.shape
    d = w1t.shape[1]
    grid = (bsz, n // R)
    return pl.pallas_call(
        functools.partial(_knn_body, n=n),
        grid=grid,
        in_specs=[
            pl.BlockSpec((1, R, 3), lambda b, r: (b, r, 0)),
            pl.BlockSpec((1, 3, n), lambda b, r: (b, 0, 0)),
            pl.BlockSpec(w1t.shape, lambda b, r: (0, 0)),
        ],
        out_specs=[
            pl.BlockSpec((1, R, K), lambda b, r: (b, r, 0)),
            pl.BlockSpec((1, R, d), lambda b, r: (b, r, 0)),
        ],
        out_shape=[
            jax.ShapeDtypeStruct((bsz, n, K), jnp.int32),
            jax.ShapeDtypeStruct((bsz, n, d), jnp.float32),
        ],
    )(xyz, xyzT, w1t)


def _sc_gather(feat_flat, a_flat, idx_flat):
    m, d = idx_flat.shape[0], feat_flat.shape[1]
    info = plsc.get_sparse_core_info()
    nw = info.num_cores * info.num_subcores
    per_w = m // nw
    nchunk = per_w // SC_CHUNK
    mesh = plsc.VectorSubcoreMesh(core_axis_name="c", subcore_axis_name="s")

    @functools.partial(
        pl.kernel,
        mesh=mesh,
        compiler_params=pltpu.CompilerParams(needs_layout_passes=False),
        out_type=(
            jax.ShapeDtypeStruct((m, d), jnp.float32),
            jax.ShapeDtypeStruct((m, d), jnp.float32),
        ),
        scratch_types=[
            pltpu.VMEM((SC_CHUNK,), jnp.int32),
            pltpu.VMEM((SC_CHUNK, d), jnp.float32),
            pltpu.VMEM((SC_CHUNK, d), jnp.float32),
            pltpu.SemaphoreType.DMA,
            pltpu.SemaphoreType.DMA,
        ],
    )
    def gather_k(feat_hbm, a_hbm, idx_hbm, fout, aout,
                 idx_v, frows, arows, sf, sa):
        wid = lax.axis_index("s") * info.num_cores + lax.axis_index("c")
        base0 = wid * per_w

        def body(i, carry):
            base = base0 + i * SC_CHUNK
            pltpu.sync_copy(idx_hbm.at[pl.ds(base, SC_CHUNK)], idx_v)
            cf = pltpu.async_copy(feat_hbm.at[idx_v], frows, sf)
            ca = pltpu.async_copy(a_hbm.at[idx_v], arows, sa)
            cf.wait()
            ca.wait()
            pltpu.sync_copy(frows, fout.at[pl.ds(base, SC_CHUNK)])
            pltpu.sync_copy(arows, aout.at[pl.ds(base, SC_CHUNK)])
            return carry

        lax.fori_loop(0, nchunk, body, 0)

    return gather_k(feat_flat, a_flat, idx_flat)


def _mlp_body(feat_ref, aq_ref, kf_ref, ka_ref,
              bd1_ref, wd2_ref, bd2_ref,
              wg1_ref, bg1_ref, wg2_ref, bg2_ref, out_ref):
    d = feat_ref.shape[-1]
    kf = kf_ref[0].reshape(R * K, d)
    ka = ka_ref[0].reshape(R * K, d)
    aq = aq_ref[0][:, None, :]
    aq = jnp.broadcast_to(aq, (R, K, d)).reshape(R * K, d)

    h = jnp.maximum((aq - ka) + bd1_ref[...], 0.0)
    de = jnp.dot(h.astype(jnp.bfloat16), wd2_ref[...],
                 preferred_element_type=jnp.float32) + bd2_ref[...]
    xk = kf + de

    gp = jnp.dot(xk.astype(jnp.bfloat16), wg1_ref[...],
                 preferred_element_type=jnp.float32) + bg1_ref[...]
    g = jnp.maximum(gp, 0.0)
    w = jnp.dot(g.astype(jnp.bfloat16), wg2_ref[...],
                preferred_element_type=jnp.float32) + bg2_ref[...]

    w3 = w.reshape(R, K, d)
    mx = jnp.max(w3, axis=1, keepdims=True)
    e = jnp.exp(w3 - mx)
    sm = e / jnp.sum(e, axis=1, keepdims=True)
    res = jnp.sum(sm * xk.reshape(R, K, d), axis=1)
    out_ref[0] = feat_ref[0] + res


def _mlp(features, aq, kfeat, ka, bd1, wd2t, bd2, wg1t, bg1, wg2t, bg2):
    bsz, n, d = features.shape
    grid = (bsz, n // R)
    return pl.pallas_call(
        _mlp_body,
        grid=grid,
        in_specs=[
            pl.BlockSpec((1, R, d), lambda b, r: (b, r, 0)),
            pl.BlockSpec((1, R, d), lambda b, r: (b, r, 0)),
            pl.BlockSpec((1, R, K, d), lambda b, r: (b, r, 0, 0)),
            pl.BlockSpec((1, R, K, d), lambda b, r: (b, r, 0, 0)),
            pl.BlockSpec(bd1.shape, lambda b, r: (0, 0)),
            pl.BlockSpec(wd2t.shape, lambda b, r: (0, 0)),
            pl.BlockSpec(bd2.shape, lambda b, r: (0, 0)),
            pl.BlockSpec(wg1t.shape, lambda b, r: (0, 0)),
            pl.BlockSpec(bg1.shape, lambda b, r: (0, 0)),
            pl.BlockSpec(wg2t.shape, lambda b, r: (0, 0)),
            pl.BlockSpec(bg2.shape, lambda b, r: (0, 0)),
        ],
        out_specs=pl.BlockSpec((1, R, d), lambda b, r: (b, r, 0)),
        out_shape=jax.ShapeDtypeStruct((bsz, n, d), jnp.float32),
    )(features, aq, kfeat, ka, bd1, wd2t, bd2, wg1t, bg1, wg2t, bg2)


NSPLIT = 4


def kernel(xyz, features, W_d1, b_d1, W_d2, b_d2, W_g1, b_g1, W_g2, b_g2):
    bsz, n, _ = xyz.shape
    d = features.shape[-1]

    w1t = W_d1.T
    bf = jnp.bfloat16
    mlp_consts = (
        b_d1.reshape(1, d), W_d2.T.astype(bf), b_d2.reshape(1, d),
        W_g1.T.astype(bf), b_g1.reshape(1, d), W_g2.T.astype(bf),
        b_g2.reshape(1, d),
    )

    bh = bsz // NSPLIT
    knns = []
    for h in range(NSPLIT):
        xyz_h = xyz[h * bh:(h + 1) * bh]
        xyzT_h = jnp.transpose(xyz_h, (0, 2, 1))
        knns.append(_knn_indices(xyz_h, xyzT_h, w1t))

    gathers = []
    for h in range(NSPLIT):
        idx, aq = knns[h]
        feat_flat = features[h * bh:(h + 1) * bh].reshape(bh * n, d)
        a_flat = aq.reshape(bh * n, d)
        gathers.append(_sc_gather(feat_flat, a_flat, idx.reshape(bh * n * K)))

    outs = []
    for h in range(NSPLIT):
        kfeat, ka = gathers[h]
        outs.append(_mlp(
            features[h * bh:(h + 1) * bh], knns[h][1],
            kfeat.reshape(bh, n, K, d), ka.reshape(bh, n, K, d), *mlp_consts))
    return jnp.concatenate(outs, axis=0)
